# baseline (speedup 1.0000x reference)
import math

import jax
import jax.numpy as jnp
import numpy as np
from jax.experimental import pallas as pl
from jax.experimental.pallas import tpu as pltpu

_NUM_SETS = 8
_SET_SIZE = 256
_NUM_DISCRETE = _NUM_SETS * _SET_SIZE
_NUM_CONT = 32
_D_MODEL = 2048
_BATCH = 4096
_BT = 256


def _threefry2x32(k0, k1, x0, x1):
    def rotl(x, d):
        return ((x << np.uint32(d)) | (x >> np.uint32(32 - d))).astype(np.uint32)

    ks0 = np.uint32(k0)
    ks1 = np.uint32(k1)
    ks2 = np.uint32(int(ks0) ^ int(ks1) ^ 0x1BD11BDA)
    ks = (ks0, ks1, ks2)
    rotations = ((13, 15, 26, 6), (17, 29, 16, 24))
    x0 = (x0 + ks0).astype(np.uint32)
    x1 = (x1 + ks1).astype(np.uint32)
    for i in range(5):
        for r in rotations[i % 2]:
            x0 = (x0 + x1).astype(np.uint32)
            x1 = rotl(x1, r)
            x1 = (x1 ^ x0).astype(np.uint32)
        x0 = (x0 + ks[(i + 1) % 3]).astype(np.uint32)
        x1 = (x1 + np.uint32((int(ks[(i + 2) % 3]) + i + 1) & 0xFFFFFFFF)
              ).astype(np.uint32)
    return x0, x1


def _random_bits(key, size):
    hi = np.zeros(size, dtype=np.uint32)
    lo = np.arange(size, dtype=np.uint32)
    o0, o1 = _threefry2x32(key[0], key[1], hi, lo)
    return o0 ^ o1


def _uniform01(bits):
    fb = (bits >> np.uint32(9)) | np.uint32(0x3F800000)
    return fb.view(np.float32) - np.float32(1.0)


def _erfinv(x):
    w = -np.log((1.0 - x) * (1.0 + x))
    y = np.empty_like(x)
    lo = w < 5.0
    ww = w[lo] - 2.5
    p = np.full_like(ww, 2.81022636e-08)
    for c in (3.43273939e-07, -3.5233877e-06, -4.39150654e-06, 0.00021858087,
              -0.00125372503, -0.00417768164, 0.246640727, 1.50140941):
        p = c + p * ww
    y[lo] = p * x[lo]
    ww = np.sqrt(w[~lo]) - 3.0
    p = np.full_like(ww, -0.000200214257)
    for c in (0.000100950558, 0.00134934322, -0.00367342844, 0.00573950773,
              -0.0076224613, 0.00943887047, 1.00167406, 2.83297682):
        p = c + p * ww
    y[~lo] = p * x[~lo]
    erf = np.frompyfunc(math.erf, 1, 1)
    k = 2.0 / math.sqrt(math.pi)
    for _ in range(2):
        y = y - (erf(y).astype(np.float64) - x) / (k * np.exp(-y * y))
    return y


def _fixed_noise():
    hi = np.zeros(2, dtype=np.uint32)
    lo = np.arange(2, dtype=np.uint32)
    o0, o1 = _threefry2x32(np.uint32(0), np.uint32(42), hi, lo)
    kg, kn = (o0[0], o1[0]), (o0[1], o1[1])

    f = _uniform01(_random_bits(kg, _BATCH * _NUM_DISCRETE))
    u = np.where(f == 0, np.float32(1e-20), f)
    g = (-np.log(-np.log(u.astype(np.float64)))).astype(np.float32)
    g = g.reshape(_BATCH, _NUM_DISCRETE)

    f = _uniform01(_random_bits(kn, _BATCH * _NUM_CONT))
    lo = np.float32(np.nextafter(np.float32(-1.0), np.float32(0.0)))
    u = np.maximum(lo, (f * np.float32(2.0) + lo).astype(np.float32))
    n = (math.sqrt(2.0) * _erfinv(u.astype(np.float64))).astype(np.float32)
    n = n.reshape(_BATCH, _NUM_CONT)
    return np.ascontiguousarray(g.T), np.ascontiguousarray(n.T)


_GUMBEL_T_NP, _NOISE_T_NP = _fixed_noise()


def _softclamp(t, value=15.0):
    return jnp.tanh(t / value) * value


def _readout_kernel(embed_ref, table_ref, gumbel_ref, noise_ref,
                    disc_ref, cont_ref):
    x = embed_ref[...]
    w = table_ref[...]
    logits_t = jax.lax.dot_general(
        w, x, (((1,), (1,)), ((), ())),
        preferred_element_type=jnp.float32,
        precision=jax.lax.Precision.DEFAULT,
    )

    noisy = logits_t[:_NUM_DISCRETE, :] + gumbel_ref[...]
    noisy3 = noisy.reshape(_NUM_SETS, _SET_SIZE, _BT)
    vmax = jnp.max(noisy3, axis=1, keepdims=True)
    iota = jax.lax.broadcasted_iota(jnp.int32, noisy3.shape, 1)
    idx = jnp.min(jnp.where(noisy3 == vmax, iota, _SET_SIZE), axis=1)
    disc_ref[...] = jnp.transpose(idx.astype(jnp.int32))

    ml = logits_t[_NUM_DISCRETE:, :]
    mu = ml[:_NUM_CONT, :]
    log_var = _softclamp(ml[_NUM_CONT:, :])
    cont_ref[...] = jnp.transpose(mu + noise_ref[...] * jnp.exp(0.5 * log_var))


def kernel(embed, embed_table):
    gumbel_t = jnp.asarray(_GUMBEL_T_NP)
    noise_t = jnp.asarray(_NOISE_T_NP)

    grid = (_BATCH // _BT,)
    disc_t, cont_t = pl.pallas_call(
        _readout_kernel,
        grid=grid,
        in_specs=[
            pl.BlockSpec((_BT, _D_MODEL), lambda i: (i, 0)),
            pl.BlockSpec((_NUM_DISCRETE + 2 * _NUM_CONT, _D_MODEL),
                         lambda i: (0, 0)),
            pl.BlockSpec((_NUM_DISCRETE, _BT), lambda i: (0, i)),
            pl.BlockSpec((_NUM_CONT, _BT), lambda i: (0, i)),
        ],
        out_specs=[
            pl.BlockSpec((_BT, _NUM_SETS), lambda i: (i, 0)),
            pl.BlockSpec((_BT, _NUM_CONT), lambda i: (i, 0)),
        ],
        out_shape=[
            jax.ShapeDtypeStruct((_BATCH, _NUM_SETS), jnp.int32),
            jax.ShapeDtypeStruct((_BATCH, _NUM_CONT), jnp.float32),
        ],
        compiler_params=pltpu.CompilerParams(
            dimension_semantics=("parallel",),
        ),
    )(embed, embed_table, gumbel_t, noise_t)
    return disc_t, cont_t


# speedup vs baseline: 1.0853x; 1.0853x over previous
import math

import jax
import jax.numpy as jnp
import numpy as np
from jax.experimental import pallas as pl
from jax.experimental.pallas import tpu as pltpu

_NUM_SETS = 8
_SET_SIZE = 256
_NUM_DISCRETE = _NUM_SETS * _SET_SIZE
_NUM_CONT = 32
_D_MODEL = 2048
_BATCH = 4096
_BT = 256


def _threefry2x32(k0, k1, x0, x1):
    def rotl(x, d):
        return ((x << np.uint32(d)) | (x >> np.uint32(32 - d))).astype(np.uint32)

    ks0 = np.uint32(k0)
    ks1 = np.uint32(k1)
    ks2 = np.uint32(int(ks0) ^ int(ks1) ^ 0x1BD11BDA)
    ks = (ks0, ks1, ks2)
    rotations = ((13, 15, 26, 6), (17, 29, 16, 24))
    x0 = (x0 + ks0).astype(np.uint32)
    x1 = (x1 + ks1).astype(np.uint32)
    for i in range(5):
        for r in rotations[i % 2]:
            x0 = (x0 + x1).astype(np.uint32)
            x1 = rotl(x1, r)
            x1 = (x1 ^ x0).astype(np.uint32)
        x0 = (x0 + ks[(i + 1) % 3]).astype(np.uint32)
        x1 = (x1 + np.uint32((int(ks[(i + 2) % 3]) + i + 1) & 0xFFFFFFFF)
              ).astype(np.uint32)
    return x0, x1


def _random_bits(key, size):
    hi = np.zeros(size, dtype=np.uint32)
    lo = np.arange(size, dtype=np.uint32)
    o0, o1 = _threefry2x32(key[0], key[1], hi, lo)
    return o0 ^ o1


def _uniform01(bits):
    fb = (bits >> np.uint32(9)) | np.uint32(0x3F800000)
    return fb.view(np.float32) - np.float32(1.0)


def _erfinv(x):
    w = -np.log((1.0 - x) * (1.0 + x))
    y = np.empty_like(x)
    lo = w < 5.0
    ww = w[lo] - 2.5
    p = np.full_like(ww, 2.81022636e-08)
    for c in (3.43273939e-07, -3.5233877e-06, -4.39150654e-06, 0.00021858087,
              -0.00125372503, -0.00417768164, 0.246640727, 1.50140941):
        p = c + p * ww
    y[lo] = p * x[lo]
    ww = np.sqrt(w[~lo]) - 3.0
    p = np.full_like(ww, -0.000200214257)
    for c in (0.000100950558, 0.00134934322, -0.00367342844, 0.00573950773,
              -0.0076224613, 0.00943887047, 1.00167406, 2.83297682):
        p = c + p * ww
    y[~lo] = p * x[~lo]
    erf = np.frompyfunc(math.erf, 1, 1)
    k = 2.0 / math.sqrt(math.pi)
    for _ in range(2):
        y = y - (erf(y).astype(np.float64) - x) / (k * np.exp(-y * y))
    return y


def _fixed_noise():
    hi = np.zeros(2, dtype=np.uint32)
    lo = np.arange(2, dtype=np.uint32)
    o0, o1 = _threefry2x32(np.uint32(0), np.uint32(42), hi, lo)
    kg, kn = (o0[0], o1[0]), (o0[1], o1[1])

    f = _uniform01(_random_bits(kg, _BATCH * _NUM_DISCRETE))
    u = np.where(f == 0, np.float32(1e-20), f)
    g = (-np.log(-np.log(u.astype(np.float64)))).astype(np.float32)
    g = g.reshape(_BATCH, _NUM_DISCRETE)

    f = _uniform01(_random_bits(kn, _BATCH * _NUM_CONT))
    lo = np.float32(np.nextafter(np.float32(-1.0), np.float32(0.0)))
    u = np.maximum(lo, (f * np.float32(2.0) + lo).astype(np.float32))
    n = (math.sqrt(2.0) * _erfinv(u.astype(np.float64))).astype(np.float32)
    n = n.reshape(_BATCH, _NUM_CONT)
    return np.ascontiguousarray(g.T), np.ascontiguousarray(n.T)


_GUMBEL_T_NP, _NOISE_T_NP = _fixed_noise()


def _softclamp(t, value=15.0):
    return jnp.tanh(t / value) * value


def _readout_kernel(embed_ref, table_ref, gumbel_ref, noise_ref,
                    disc_ref, cont_ref, wbf_ref):
    @pl.when(pl.program_id(0) == 0)
    def _():
        wbf_ref[...] = table_ref[...].astype(jnp.bfloat16)

    x = embed_ref[...].astype(jnp.bfloat16)
    w = wbf_ref[...]
    logits_t = jax.lax.dot_general(
        w, x, (((1,), (1,)), ((), ())),
        preferred_element_type=jnp.float32,
        precision=jax.lax.Precision.DEFAULT,
    )

    noisy = logits_t[:_NUM_DISCRETE, :] + gumbel_ref[...]
    noisy3 = noisy.reshape(_NUM_SETS, _SET_SIZE, _BT)
    vmax = jnp.max(noisy3, axis=1, keepdims=True)
    iota = jax.lax.broadcasted_iota(jnp.int32, noisy3.shape, 1)
    idx = jnp.min(jnp.where(noisy3 == vmax, iota, _SET_SIZE), axis=1)
    disc_ref[...] = idx.astype(jnp.int32)

    ml = logits_t[_NUM_DISCRETE:, :]
    mu = ml[:_NUM_CONT, :]
    log_var = _softclamp(ml[_NUM_CONT:, :])
    cont_ref[...] = mu + noise_ref[...] * jnp.exp(0.5 * log_var)


def kernel(embed, embed_table):
    gumbel_t = jnp.asarray(_GUMBEL_T_NP)
    noise_t = jnp.asarray(_NOISE_T_NP)

    grid = (_BATCH // _BT,)
    disc_t, cont_t = pl.pallas_call(
        _readout_kernel,
        grid=grid,
        in_specs=[
            pl.BlockSpec((_BT, _D_MODEL), lambda i: (i, 0)),
            pl.BlockSpec((_NUM_DISCRETE + 2 * _NUM_CONT, _D_MODEL),
                         lambda i: (0, 0)),
            pl.BlockSpec((_NUM_DISCRETE, _BT), lambda i: (0, i)),
            pl.BlockSpec((_NUM_CONT, _BT), lambda i: (0, i)),
        ],
        out_specs=[
            pl.BlockSpec((_NUM_SETS, _BT), lambda i: (0, i)),
            pl.BlockSpec((_NUM_CONT, _BT), lambda i: (0, i)),
        ],
        out_shape=[
            jax.ShapeDtypeStruct((_NUM_SETS, _BATCH), jnp.int32),
            jax.ShapeDtypeStruct((_NUM_CONT, _BATCH), jnp.float32),
        ],
        scratch_shapes=[
            pltpu.VMEM((_NUM_DISCRETE + 2 * _NUM_CONT, _D_MODEL), jnp.bfloat16),
        ],
        compiler_params=pltpu.CompilerParams(
            dimension_semantics=("parallel",),
        ),
    )(embed, embed_table, gumbel_t, noise_t)
    return disc_t.T, cont_t.T


# speedup vs baseline: 1.1446x; 1.0546x over previous
import math

import jax
import jax.numpy as jnp
import numpy as np
from jax.experimental import pallas as pl
from jax.experimental.pallas import tpu as pltpu

_NUM_SETS = 8
_SET_SIZE = 256
_NUM_DISCRETE = _NUM_SETS * _SET_SIZE
_NUM_CONT = 32
_D_MODEL = 2048
_BATCH = 4096
_BT = 256


def _threefry2x32(k0, k1, x0, x1):
    def rotl(x, d):
        return ((x << np.uint32(d)) | (x >> np.uint32(32 - d))).astype(np.uint32)

    ks0 = np.uint32(k0)
    ks1 = np.uint32(k1)
    ks2 = np.uint32(int(ks0) ^ int(ks1) ^ 0x1BD11BDA)
    ks = (ks0, ks1, ks2)
    rotations = ((13, 15, 26, 6), (17, 29, 16, 24))
    x0 = (x0 + ks0).astype(np.uint32)
    x1 = (x1 + ks1).astype(np.uint32)
    for i in range(5):
        for r in rotations[i % 2]:
            x0 = (x0 + x1).astype(np.uint32)
            x1 = rotl(x1, r)
            x1 = (x1 ^ x0).astype(np.uint32)
        x0 = (x0 + ks[(i + 1) % 3]).astype(np.uint32)
        x1 = (x1 + np.uint32((int(ks[(i + 2) % 3]) + i + 1) & 0xFFFFFFFF)
              ).astype(np.uint32)
    return x0, x1


def _random_bits(key, size):
    hi = np.zeros(size, dtype=np.uint32)
    lo = np.arange(size, dtype=np.uint32)
    o0, o1 = _threefry2x32(key[0], key[1], hi, lo)
    return o0 ^ o1


def _uniform01(bits):
    fb = (bits >> np.uint32(9)) | np.uint32(0x3F800000)
    return fb.view(np.float32) - np.float32(1.0)


def _erfinv(x):
    w = -np.log((1.0 - x) * (1.0 + x))
    y = np.empty_like(x)
    lo = w < 5.0
    ww = w[lo] - 2.5
    p = np.full_like(ww, 2.81022636e-08)
    for c in (3.43273939e-07, -3.5233877e-06, -4.39150654e-06, 0.00021858087,
              -0.00125372503, -0.00417768164, 0.246640727, 1.50140941):
        p = c + p * ww
    y[lo] = p * x[lo]
    ww = np.sqrt(w[~lo]) - 3.0
    p = np.full_like(ww, -0.000200214257)
    for c in (0.000100950558, 0.00134934322, -0.00367342844, 0.00573950773,
              -0.0076224613, 0.00943887047, 1.00167406, 2.83297682):
        p = c + p * ww
    y[~lo] = p * x[~lo]
    erf = np.frompyfunc(math.erf, 1, 1)
    k = 2.0 / math.sqrt(math.pi)
    for _ in range(2):
        y = y - (erf(y).astype(np.float64) - x) / (k * np.exp(-y * y))
    return y


def _fixed_noise():
    hi = np.zeros(2, dtype=np.uint32)
    lo = np.arange(2, dtype=np.uint32)
    o0, o1 = _threefry2x32(np.uint32(0), np.uint32(42), hi, lo)
    kg, kn = (o0[0], o1[0]), (o0[1], o1[1])

    f = _uniform01(_random_bits(kg, _BATCH * _NUM_DISCRETE))
    u = np.where(f == 0, np.float32(1e-20), f)
    g = (-np.log(-np.log(u.astype(np.float64)))).astype(np.float32)
    g = g.reshape(_BATCH, _NUM_DISCRETE)

    f = _uniform01(_random_bits(kn, _BATCH * _NUM_CONT))
    lo = np.float32(np.nextafter(np.float32(-1.0), np.float32(0.0)))
    u = np.maximum(lo, (f * np.float32(2.0) + lo).astype(np.float32))
    n = (math.sqrt(2.0) * _erfinv(u.astype(np.float64))).astype(np.float32)
    n = n.reshape(_BATCH, _NUM_CONT)
    return np.ascontiguousarray(g.T), np.ascontiguousarray(n.T)


_GUMBEL_T_NP, _NOISE_T_NP = _fixed_noise()


def _softclamp(t, value=15.0):
    return jnp.tanh(t / value) * value


def _readout_kernel(embed_ref, table_a_ref, table_b_ref, gumbel_ref,
                    noise_ref, disc_ref, cont_ref):
    x = embed_ref[...]
    xa = x[:, : _D_MODEL // 2]
    xb = x[:, _D_MODEL // 2:]
    dn = (((1,), (1,)), ((), ()))
    logits_t = jax.lax.dot_general(
        table_a_ref[...], xa, dn,
        preferred_element_type=jnp.float32,
        precision=jax.lax.Precision.DEFAULT,
    ) + jax.lax.dot_general(
        table_b_ref[...], xb, dn,
        preferred_element_type=jnp.float32,
        precision=jax.lax.Precision.DEFAULT,
    )

    noisy = logits_t[:_NUM_DISCRETE, :] + gumbel_ref[...]
    noisy3 = noisy.reshape(_NUM_SETS, _SET_SIZE, _BT)
    vmax = jnp.max(noisy3, axis=1, keepdims=True)
    iota = jax.lax.broadcasted_iota(jnp.int32, noisy3.shape, 1)
    idx = jnp.min(jnp.where(noisy3 == vmax, iota, _SET_SIZE), axis=1)
    disc_ref[...] = idx.astype(jnp.int32)

    ml = logits_t[_NUM_DISCRETE:, :]
    mu = ml[:_NUM_CONT, :]
    log_var = _softclamp(ml[_NUM_CONT:, :])
    cont_ref[...] = mu + noise_ref[...] * jnp.exp(0.5 * log_var)


def kernel(embed, embed_table):
    gumbel_t = jnp.asarray(_GUMBEL_T_NP)
    noise_t = jnp.asarray(_NOISE_T_NP)

    grid = (_BATCH // _BT,)
    disc_t, cont_t = pl.pallas_call(
        _readout_kernel,
        grid=grid,
        in_specs=[
            pl.BlockSpec((_BT, _D_MODEL), lambda i: (i, 0)),
            pl.BlockSpec((_NUM_DISCRETE + 2 * _NUM_CONT, _D_MODEL // 2),
                         lambda i: (0, 0)),
            pl.BlockSpec((_NUM_DISCRETE + 2 * _NUM_CONT, _D_MODEL // 2),
                         lambda i: (0, 1)),
            pl.BlockSpec((_NUM_DISCRETE, _BT), lambda i: (0, i)),
            pl.BlockSpec((_NUM_CONT, _BT), lambda i: (0, i)),
        ],
        out_specs=[
            pl.BlockSpec((_NUM_SETS, _BT), lambda i: (0, i)),
            pl.BlockSpec((_NUM_CONT, _BT), lambda i: (0, i)),
        ],
        out_shape=[
            jax.ShapeDtypeStruct((_NUM_SETS, _BATCH), jnp.int32),
            jax.ShapeDtypeStruct((_NUM_CONT, _BATCH), jnp.float32),
        ],
        compiler_params=pltpu.CompilerParams(
            dimension_semantics=("parallel",),
        ),
    )(embed, embed_table, embed_table, gumbel_t, noise_t)
    return disc_t.T, cont_t.T
